# baseline (speedup 1.0000x reference)
import functools

import jax
import jax.numpy as jnp
from jax import lax
from jax.experimental import pallas as pl
from jax.experimental.pallas import tpu as pltpu
from jax.experimental.pallas import tpu_sc as plsc

SEQ = 4096
EMB = 1024
EPS = 1e-5
L = 16
NC, NS = 2, 16
NW = NC * NS
RPW = SEQ // NW
G = L
NG = RPW // G

_mesh = plsc.VectorSubcoreMesh(core_axis_name="c", subcore_axis_name="s")


def _rsqrt(v):
    i = plsc.bitcast(v, jnp.int32)
    i = jnp.int32(0x5F3759DF) - lax.shift_right_logical(i, 1)
    y = plsc.bitcast(i, jnp.float32)
    for _ in range(3):
        y = y * (1.5 - 0.5 * v * y * y)
    return y


@functools.partial(
    pl.kernel,
    out_type=jax.ShapeDtypeStruct((SEQ, EMB), jnp.float32),
    mesh=_mesh,
    compiler_params=pltpu.CompilerParams(use_tc_tiling_on_sc=False,
                                         needs_layout_passes=False),
    scratch_types=[
        pltpu.VMEM((G,), jnp.int32),
        pltpu.VMEM((G,), jnp.int32),
        pltpu.VMEM((G, EMB), jnp.float32),
        pltpu.VMEM((G, EMB), jnp.float32),
        pltpu.VMEM((EMB,), jnp.float32),
        pltpu.VMEM((EMB,), jnp.float32),
        pltpu.VMEM((2, EMB), jnp.float32),
        pltpu.SemaphoreType.DMA,
    ],
)
def _sc_embedder(tok_hbm, seg_hbm, ttab_hbm, stab_hbm, ptab_hbm, w_hbm,
                 b_hbm, out_hbm, idx_v, sid_v, xbuf, pbuf, w_v, b_v,
                 stab_v, sem):
    wid = lax.axis_index("s") * NC + lax.axis_index("c")
    base = wid * RPW
    pltpu.sync_copy(w_hbm, w_v)
    pltpu.sync_copy(b_hbm, b_v)
    pltpu.sync_copy(stab_hbm, stab_v)
    rows = lax.iota(jnp.int32, L)

    def group(g, _):
        rbase = base + g * G
        pltpu.sync_copy(tok_hbm.at[pl.ds(rbase, G)], idx_v)
        pltpu.sync_copy(seg_hbm.at[pl.ds(rbase, G)], sid_v)
        pltpu.async_copy(ttab_hbm.at[idx_v], xbuf, sem).wait()
        pltpu.sync_copy(ptab_hbm.at[pl.ds(rbase, G)], pbuf)
        sidv = sid_v[...]

        def pass1(d, carry):
            acc, acc2 = carry
            dd = jnp.full((L,), d, jnp.int32)
            tv = plsc.load_gather(xbuf, [rows, dd])
            pv = plsc.load_gather(pbuf, [rows, dd])
            sv = plsc.load_gather(stab_v, [sidv, dd])
            x = tv + pv + sv
            plsc.store_scatter(xbuf, [rows, dd], x)
            return acc + x, acc2 + x * x

        zero = jnp.zeros((L,), jnp.float32)
        acc, acc2 = lax.fori_loop(0, EMB, pass1, (zero, zero), unroll=4)
        mu = acc * (1.0 / EMB)
        var = acc2 * (1.0 / EMB) - mu * mu
        rs = _rsqrt(var + EPS)

        def pass2(d, _):
            dd = jnp.full((L,), d, jnp.int32)
            x = plsc.load_gather(xbuf, [rows, dd])
            wg = plsc.load_gather(w_v, [dd])
            bg = plsc.load_gather(b_v, [dd])
            y = (x - mu) * rs * wg + bg
            plsc.store_scatter(xbuf, [rows, dd], y)
            return 0

        lax.fori_loop(0, EMB, pass2, 0, unroll=4)
        pltpu.sync_copy(xbuf, out_hbm.at[pl.ds(rbase, G)])
        return 0

    lax.fori_loop(0, NG, group, 0)


def kernel(token_ids, position_ids, segment_ids, token_table, segment_table,
           position_table, ln_weight, ln_bias):
    del position_ids
    return _sc_embedder(token_ids.astype(jnp.int32),
                        segment_ids.astype(jnp.int32), token_table,
                        segment_table, position_table, ln_weight, ln_bias)


# speedup vs baseline: 1.3699x; 1.3699x over previous
import functools

import jax
import jax.numpy as jnp
from jax import lax
from jax.experimental import pallas as pl
from jax.experimental.pallas import tpu as pltpu
from jax.experimental.pallas import tpu_sc as plsc

SEQ = 4096
EMB = 1024
EPS = 1e-5
L = 16
NC, NS = 2, 16
NW = NC * NS
RPW = SEQ // NW
G = L
NG = RPW // G

_mesh = plsc.VectorSubcoreMesh(core_axis_name="c", subcore_axis_name="s")


def _rsqrt(v):
    i = plsc.bitcast(v, jnp.int32)
    i = jnp.int32(0x5F3759DF) - lax.shift_right_logical(i, 1)
    y = plsc.bitcast(i, jnp.float32)
    for _ in range(3):
        y = y * (1.5 - 0.5 * v * y * y)
    return y


@functools.partial(
    pl.kernel,
    out_type=jax.ShapeDtypeStruct((SEQ, EMB), jnp.float32),
    mesh=_mesh,
    compiler_params=pltpu.CompilerParams(use_tc_tiling_on_sc=False,
                                         needs_layout_passes=False),
    scratch_types=[
        pltpu.VMEM((G,), jnp.int32),
        pltpu.VMEM((G,), jnp.int32),
        pltpu.VMEM((G, EMB), jnp.float32),
        pltpu.VMEM((G, EMB), jnp.float32),
        pltpu.VMEM((G, EMB), jnp.float32),
        pltpu.VMEM((EMB,), jnp.float32),
        pltpu.VMEM((EMB,), jnp.float32),
        pltpu.VMEM((2, EMB), jnp.float32),
        pltpu.SemaphoreType.DMA,
    ],
)
def _sc_embedder(tok_hbm, seg_hbm, ttab_hbm, stab_hbm, ptab_hbm, w_hbm,
                 b_hbm, out_hbm, idx_v, sid_v, tbuf, pbuf, xbuf, w_v, b_v,
                 stab_v, sem):
    wid = lax.axis_index("s") * NC + lax.axis_index("c")
    base = wid * RPW
    pltpu.sync_copy(w_hbm, w_v)
    pltpu.sync_copy(b_hbm, b_v)
    pltpu.sync_copy(stab_hbm, stab_v)
    rows = lax.iota(jnp.int32, L)

    def group(g, _):
        rbase = base + g * G
        pltpu.sync_copy(tok_hbm.at[pl.ds(rbase, G)], idx_v)
        pltpu.sync_copy(seg_hbm.at[pl.ds(rbase, G)], sid_v)
        pltpu.async_copy(ttab_hbm.at[idx_v], tbuf, sem).wait()
        pltpu.sync_copy(ptab_hbm.at[pl.ds(rbase, G)], pbuf)
        sidv = sid_v[...]
        zero = jnp.zeros((L,), jnp.float32)

        @plsc.parallel_loop(0, EMB, unroll=8, carry=(zero, zero))
        def pass1(d, carry):
            acc, acc2 = carry
            dd = jnp.full((L,), d, jnp.int32)
            tv = plsc.load_gather(tbuf, [rows, dd])
            pv = plsc.load_gather(pbuf, [rows, dd])
            sv = plsc.load_gather(stab_v, [sidv, dd])
            x = tv + pv + sv
            plsc.store_scatter(xbuf, [rows, dd], x)
            return acc + x, acc2 + x * x

        acc, acc2 = pass1
        mu = acc * (1.0 / EMB)
        var = acc2 * (1.0 / EMB) - mu * mu
        rs = _rsqrt(var + EPS)

        @plsc.parallel_loop(0, EMB, unroll=8)
        def pass2(d):
            dd = jnp.full((L,), d, jnp.int32)
            x = plsc.load_gather(xbuf, [rows, dd])
            wg = plsc.load_gather(w_v, [dd])
            bg = plsc.load_gather(b_v, [dd])
            y = (x - mu) * rs * wg + bg
            plsc.store_scatter(pbuf, [rows, dd], y)

        pltpu.sync_copy(pbuf, out_hbm.at[pl.ds(rbase, G)])
        return 0

    lax.fori_loop(0, NG, group, 0)


def kernel(token_ids, position_ids, segment_ids, token_table, segment_table,
           position_table, ln_weight, ln_bias):
    del position_ids
    return _sc_embedder(token_ids.astype(jnp.int32),
                        segment_ids.astype(jnp.int32), token_table,
                        segment_table, position_table, ln_weight, ln_bias)


# speedup vs baseline: 2.1646x; 1.5802x over previous
import functools

import jax
import jax.numpy as jnp
from jax import lax
from jax.experimental import pallas as pl
from jax.experimental.pallas import tpu as pltpu
from jax.experimental.pallas import tpu_sc as plsc

SEQ = 4096
EMB = 1024
EPS = 1e-5
L = 16
NC, NS = 2, 16
NW = NC * NS
RPW = SEQ // NW
G = L
NG = RPW // G

_mesh = plsc.VectorSubcoreMesh(core_axis_name="c", subcore_axis_name="s")


def _rsqrt(v):
    i = plsc.bitcast(v, jnp.int32)
    i = jnp.int32(0x5F3759DF) - lax.shift_right_logical(i, 1)
    y = plsc.bitcast(i, jnp.float32)
    for _ in range(3):
        y = y * (1.5 - 0.5 * v * y * y)
    return y


@functools.partial(
    pl.kernel,
    out_type=jax.ShapeDtypeStruct((SEQ, EMB), jnp.float32),
    mesh=_mesh,
    compiler_params=pltpu.CompilerParams(use_tc_tiling_on_sc=False,
                                         needs_layout_passes=False),
    scratch_types=[
        pltpu.VMEM((G,), jnp.int32),
        pltpu.VMEM((G,), jnp.int32),
        pltpu.VMEM((G, EMB), jnp.float32),
        pltpu.VMEM((G, EMB), jnp.float32),
        pltpu.VMEM((G, EMB), jnp.float32),
        pltpu.VMEM((EMB,), jnp.float32),
        pltpu.VMEM((EMB,), jnp.float32),
        pltpu.VMEM((2, EMB), jnp.float32),
        pltpu.SemaphoreType.DMA,
    ],
)
def _sc_embedder(tok_hbm, seg_hbm, ttab_hbm, stab_hbm, ptab_hbm, w_hbm,
                 b_hbm, out_hbm, idx_v, sid_v, tbuf, pbuf, xbuf, w_v, b_v,
                 stab_v, sem):
    wid = lax.axis_index("s") * NC + lax.axis_index("c")
    base = wid * RPW
    pltpu.sync_copy(w_hbm, w_v)
    pltpu.sync_copy(b_hbm, b_v)
    pltpu.sync_copy(stab_hbm, stab_v)
    rows = lax.iota(jnp.int32, L)

    def group(g, _):
        rbase = base + g * G
        pltpu.sync_copy(tok_hbm.at[pl.ds(rbase, G)], idx_v)
        pltpu.sync_copy(seg_hbm.at[pl.ds(rbase, G)], sid_v)
        pltpu.async_copy(ttab_hbm.at[idx_v], tbuf, sem).wait()
        pltpu.sync_copy(ptab_hbm.at[pl.ds(rbase, G)], pbuf)
        sidv = sid_v[...]
        zero = jnp.zeros((L,), jnp.float32)
        NACC = 4

        @plsc.parallel_loop(0, EMB, step=NACC, unroll=2,
                            carry=tuple(zero for _ in range(2 * NACC)))
        def pass1(d, carry):
            out = []
            for j in range(NACC):
                acc, acc2 = carry[2 * j], carry[2 * j + 1]
                dd = (rows + (d + j)) & (EMB - 1)
                tv = plsc.load_gather(tbuf, [rows, dd])
                pv = plsc.load_gather(pbuf, [rows, dd])
                sv = plsc.load_gather(stab_v, [sidv, dd])
                x = tv + pv + sv
                plsc.store_scatter(xbuf, [rows, dd], x)
                out += [acc + x, acc2 + x * x]
            return tuple(out)

        acc = pass1[0] + pass1[2] + pass1[4] + pass1[6]
        acc2 = pass1[1] + pass1[3] + pass1[5] + pass1[7]
        mu = acc * (1.0 / EMB)
        var = acc2 * (1.0 / EMB) - mu * mu
        rs = _rsqrt(var + EPS)

        @plsc.parallel_loop(0, EMB, step=NACC, unroll=2)
        def pass2(d):
            for j in range(NACC):
                dd = (rows + (d + j)) & (EMB - 1)
                x = plsc.load_gather(xbuf, [rows, dd])
                wg = plsc.load_gather(w_v, [dd])
                bg = plsc.load_gather(b_v, [dd])
                y = (x - mu) * rs * wg + bg
                plsc.store_scatter(pbuf, [rows, dd], y)

        pltpu.sync_copy(pbuf, out_hbm.at[pl.ds(rbase, G)])
        return 0

    lax.fori_loop(0, NG, group, 0)


def kernel(token_ids, position_ids, segment_ids, token_table, segment_table,
           position_table, ln_weight, ln_bias):
    del position_ids
    return _sc_embedder(token_ids.astype(jnp.int32),
                        segment_ids.astype(jnp.int32), token_table,
                        segment_table, position_table, ln_weight, ln_bias)


# speedup vs baseline: 8.9698x; 4.1438x over previous
import functools

import jax
import jax.numpy as jnp
from jax import lax
from jax.experimental import pallas as pl
from jax.experimental.pallas import tpu as pltpu
from jax.experimental.pallas import tpu_sc as plsc

SEQ = 4096
EMB = 1024
EPS = 1e-5
L = 16
NC, NS = 2, 16
NW = NC * NS
RPW = SEQ // NW
G = L
NG = RPW // G

_mesh = plsc.VectorSubcoreMesh(core_axis_name="c", subcore_axis_name="s")


def _rsqrt(v):
    i = plsc.bitcast(v, jnp.int32)
    i = jnp.int32(0x5F3759DF) - lax.shift_right_logical(i, 1)
    y = plsc.bitcast(i, jnp.float32)
    for _ in range(3):
        y = y * (1.5 - 0.5 * v * y * y)
    return y


@functools.partial(
    pl.kernel,
    out_type=jax.ShapeDtypeStruct((SEQ, EMB), jnp.float32),
    mesh=_mesh,
    compiler_params=pltpu.CompilerParams(use_tc_tiling_on_sc=True,
                                         needs_layout_passes=False),
    scratch_types=[
        pltpu.VMEM((G,), jnp.int32),
        pltpu.VMEM((G,), jnp.int32),
        pltpu.VMEM((G, EMB), jnp.float32),
        pltpu.VMEM((G, EMB), jnp.float32),
        pltpu.VMEM((G, EMB), jnp.float32),
        pltpu.VMEM((EMB,), jnp.float32),
        pltpu.VMEM((EMB,), jnp.float32),
        pltpu.VMEM((2, EMB), jnp.float32),
        pltpu.SemaphoreType.DMA,
    ],
)
def _sc_embedder(tok_hbm, seg_hbm, ttab_hbm, stab_hbm, ptab_hbm, w_hbm,
                 b_hbm, out_hbm, idx_v, sid_v, tbuf, pbuf, xbuf, w_v, b_v,
                 stab_v, sem):
    wid = lax.axis_index("s") * NC + lax.axis_index("c")
    base = wid * RPW
    pltpu.sync_copy(w_hbm, w_v)
    pltpu.sync_copy(b_hbm, b_v)
    pltpu.sync_copy(stab_hbm, stab_v)
    rows = lax.iota(jnp.int32, L)

    def group(g, _):
        rbase = base + g * G
        pltpu.sync_copy(tok_hbm.at[pl.ds(rbase, G)], idx_v)
        pltpu.sync_copy(seg_hbm.at[pl.ds(rbase, G)], sid_v)
        pltpu.async_copy(ttab_hbm.at[idx_v], tbuf, sem).wait()
        pltpu.sync_copy(ptab_hbm.at[pl.ds(rbase, G)], pbuf)
        sidv = sid_v[...]
        zero = jnp.zeros((L,), jnp.float32)
        NACC = 4

        @plsc.parallel_loop(0, EMB, step=NACC, unroll=2,
                            carry=tuple(zero for _ in range(2 * NACC)))
        def pass1(d, carry):
            out = []
            for j in range(NACC):
                acc, acc2 = carry[2 * j], carry[2 * j + 1]
                dd = (rows + (d + j)) & (EMB - 1)
                tv = plsc.load_gather(tbuf, [rows, dd])
                pv = plsc.load_gather(pbuf, [rows, dd])
                sv = plsc.load_gather(stab_v, [sidv, dd])
                x = tv + pv + sv
                plsc.store_scatter(xbuf, [rows, dd], x)
                out += [acc + x, acc2 + x * x]
            return tuple(out)

        acc = pass1[0] + pass1[2] + pass1[4] + pass1[6]
        acc2 = pass1[1] + pass1[3] + pass1[5] + pass1[7]
        mu = acc * (1.0 / EMB)
        var = acc2 * (1.0 / EMB) - mu * mu
        rs = _rsqrt(var + EPS)

        @plsc.parallel_loop(0, EMB, step=NACC, unroll=2)
        def pass2(d):
            for j in range(NACC):
                dd = (rows + (d + j)) & (EMB - 1)
                x = plsc.load_gather(xbuf, [rows, dd])
                wg = plsc.load_gather(w_v, [dd])
                bg = plsc.load_gather(b_v, [dd])
                y = (x - mu) * rs * wg + bg
                plsc.store_scatter(pbuf, [rows, dd], y)

        pltpu.sync_copy(pbuf, out_hbm.at[pl.ds(rbase, G)])
        return 0

    lax.fori_loop(0, NG, group, 0)


def kernel(token_ids, position_ids, segment_ids, token_table, segment_table,
           position_table, ln_weight, ln_bias):
    del position_ids
    return _sc_embedder(token_ids.astype(jnp.int32),
                        segment_ids.astype(jnp.int32), token_table,
                        segment_table, position_table, ln_weight, ln_bias)


# speedup vs baseline: 10.5964x; 1.1813x over previous
import functools

import jax
import jax.numpy as jnp
from jax import lax
from jax.experimental import pallas as pl
from jax.experimental.pallas import tpu as pltpu
from jax.experimental.pallas import tpu_sc as plsc

SEQ = 4096
EMB = 1024
EPS = 1e-5
L = 16
NC, NS = 2, 16
NW = NC * NS
RPW = SEQ // NW
G = L
NG = RPW // G

_mesh = plsc.VectorSubcoreMesh(core_axis_name="c", subcore_axis_name="s")


def _rsqrt(v):
    i = plsc.bitcast(v, jnp.int32)
    i = jnp.int32(0x5F3759DF) - lax.shift_right_logical(i, 1)
    y = plsc.bitcast(i, jnp.float32)
    for _ in range(3):
        y = y * (1.5 - 0.5 * v * y * y)
    return y


@functools.partial(
    pl.kernel,
    out_type=jax.ShapeDtypeStruct((SEQ, EMB), jnp.float32),
    mesh=_mesh,
    compiler_params=pltpu.CompilerParams(use_tc_tiling_on_sc=True,
                                         needs_layout_passes=False),
    scratch_types=[
        [pltpu.VMEM((G,), jnp.int32)] * 2,
        [pltpu.VMEM((G,), jnp.int32)] * 2,
        [pltpu.VMEM((G, EMB), jnp.float32)] * 2,
        [pltpu.VMEM((G, EMB), jnp.float32)] * 2,
        pltpu.VMEM((G, EMB), jnp.float32),
        pltpu.VMEM((EMB,), jnp.float32),
        pltpu.VMEM((EMB,), jnp.float32),
        pltpu.VMEM((2, EMB), jnp.float32),
        [pltpu.SemaphoreType.DMA] * 2,
        [pltpu.SemaphoreType.DMA] * 2,
        [pltpu.SemaphoreType.DMA] * 2,
    ],
)
def _sc_embedder(tok_hbm, seg_hbm, ttab_hbm, stab_hbm, ptab_hbm, w_hbm,
                 b_hbm, out_hbm, idx_v, sid_v, tbuf, pbuf, xbuf, w_v, b_v,
                 stab_v, gsem, psem, osem):
    wid = lax.axis_index("s") * NC + lax.axis_index("c")
    base = wid * RPW
    pltpu.sync_copy(w_hbm, w_v)
    pltpu.sync_copy(b_hbm, b_v)
    pltpu.sync_copy(stab_hbm, stab_v)
    rows = lax.iota(jnp.int32, L)

    hdl_g = [None, None]
    hdl_p = [None, None]
    hdl_o = [None, None]

    def start_group(g):
        b = g % 2
        rbase = base + g * G
        pltpu.sync_copy(tok_hbm.at[pl.ds(rbase, G)], idx_v[b])
        pltpu.sync_copy(seg_hbm.at[pl.ds(rbase, G)], sid_v[b])
        hdl_g[b] = pltpu.async_copy(ttab_hbm.at[idx_v[b]], tbuf[b], gsem[b])
        hdl_p[b] = pltpu.async_copy(ptab_hbm.at[pl.ds(rbase, G)], pbuf[b],
                                    psem[b])

    start_group(0)
    for g in range(NG):
        b = g % 2
        rbase = base + g * G
        if g + 1 < NG:
            if hdl_o[1 - b] is not None:
                hdl_o[1 - b].wait()
            start_group(g + 1)
        hdl_g[b].wait()
        hdl_p[b].wait()
        _ln_group(rows, sid_v[b], tbuf[b], pbuf[b], xbuf, w_v, b_v, stab_v)
        hdl_o[b] = pltpu.async_copy(pbuf[b], out_hbm.at[pl.ds(rbase, G)],
                                    osem[b])
    hdl_o[0].wait()
    hdl_o[1].wait()


def _ln_group(rows, sid_ref, tbuf, pbuf, xbuf, w_v, b_v, stab_v):
        sidv = sid_ref[...]
        zero = jnp.zeros((L,), jnp.float32)
        NACC = 4

        @plsc.parallel_loop(0, EMB, step=NACC, unroll=2,
                            carry=tuple(zero for _ in range(2 * NACC)))
        def pass1(d, carry):
            out = []
            for j in range(NACC):
                acc, acc2 = carry[2 * j], carry[2 * j + 1]
                dd = (rows + (d + j)) & (EMB - 1)
                tv = plsc.load_gather(tbuf, [rows, dd])
                pv = plsc.load_gather(pbuf, [rows, dd])
                sv = plsc.load_gather(stab_v, [sidv, dd])
                x = tv + pv + sv
                plsc.store_scatter(xbuf, [rows, dd], x)
                out += [acc + x, acc2 + x * x]
            return tuple(out)

        acc = pass1[0] + pass1[2] + pass1[4] + pass1[6]
        acc2 = pass1[1] + pass1[3] + pass1[5] + pass1[7]
        mu = acc * (1.0 / EMB)
        var = acc2 * (1.0 / EMB) - mu * mu
        rs = _rsqrt(var + EPS)

        @plsc.parallel_loop(0, EMB, step=NACC, unroll=2)
        def pass2(d):
            for j in range(NACC):
                dd = (rows + (d + j)) & (EMB - 1)
                x = plsc.load_gather(xbuf, [rows, dd])
                wg = plsc.load_gather(w_v, [dd])
                bg = plsc.load_gather(b_v, [dd])
                y = (x - mu) * rs * wg + bg
                plsc.store_scatter(pbuf, [rows, dd], y)


def kernel(token_ids, position_ids, segment_ids, token_table, segment_table,
           position_table, ln_weight, ln_bias):
    del position_ids
    return _sc_embedder(token_ids.astype(jnp.int32),
                        segment_ids.astype(jnp.int32), token_table,
                        segment_table, position_table, ln_weight, ln_bias)
